# baseline (speedup 1.0000x reference)
import functools

import jax
import jax.numpy as jnp
from jax.experimental import pallas as pl
from jax.experimental.pallas import tpu as pltpu


def _round_up(x, m):
    return ((x + m - 1) // m) * m


def _encode2_kernel(x_ref, w1e_ref, b1e_ref, whe_ref, bhe_ref,
                    w1c_ref, b1c_ref, whc_ref, bhc_ref,
                    mu_ref, sg_ref, ce_ref, cmu_ref, cc_ref, *, L_pad):
    x = x_ref[...].astype(jnp.bfloat16)

    def heads(w1_ref, b1_ref, wh_ref, bh_ref):
        h = jnp.dot(x, w1_ref[...], preferred_element_type=jnp.float32)
        h = jnp.maximum(h + b1_ref[...], 0.0).astype(jnp.bfloat16)
        head = jnp.dot(h, wh_ref[...], preferred_element_type=jnp.float32)
        head = head + bh_ref[...]
        mu = head[:, :L_pad]
        sg = head[:, L_pad:]
        elu = jnp.where(sg > 0, sg, jnp.exp(jnp.minimum(sg, 0.0)) - 1.0)
        return mu, elu + 1.0 + 1e-14

    mu_e, sig_e = heads(w1e_ref, b1e_ref, whe_ref, bhe_ref)
    mu_ref[...] = mu_e
    sg_ref[...] = sig_e
    ce_ref[:, 0, :] = mu_e
    ce_ref[:, 1, :] = sig_e

    mu_c, sig_c = heads(w1c_ref, b1c_ref, whc_ref, bhc_ref)
    cmu_ref[...] = mu_c
    cc_ref[:, 0, :] = mu_c
    cc_ref[:, 1, :] = sig_c


def _encode_both(X, enc_w, ctx_w, *, tile_n=512):
    N, D = X.shape
    H = enc_w[0].shape[1]
    L = enc_w[2].shape[1]
    L_pad = _round_up(L, 128)
    W2 = 2 * L_pad

    def head_operands(W1, b1, Wmu, bmu, Wsg, bsg):
        Whead = jnp.zeros((H, W2), jnp.float32)
        Whead = Whead.at[:, :L].set(Wmu)
        Whead = Whead.at[:, L_pad:L_pad + L].set(Wsg)
        bhead = jnp.zeros((1, W2), jnp.float32)
        bhead = bhead.at[0, :L].set(bmu)
        bhead = bhead.at[0, L_pad:L_pad + L].set(bsg)
        return (W1.astype(jnp.bfloat16), b1.reshape(1, H),
                Whead.astype(jnp.bfloat16), bhead)

    enc_ops = head_operands(*enc_w)
    ctx_ops = head_operands(*ctx_w)

    tile_n = min(tile_n, _round_up(N, 8))
    N_pad = _round_up(N, tile_n)
    Xp = X if N_pad == N else jnp.pad(X, ((0, N_pad - N), (0, 0)))

    grid = (N_pad // tile_n,)
    inv = lambda shape: pl.BlockSpec(shape, lambda i: (0, 0))
    row = lambda w: pl.BlockSpec((tile_n, w), lambda i: (i, 0))

    row3 = pl.BlockSpec((tile_n, 2, L_pad), lambda i: (i, 0, 0))
    mu, sigma, ce, cmu, cc = pl.pallas_call(
        functools.partial(_encode2_kernel, L_pad=L_pad),
        out_shape=(jax.ShapeDtypeStruct((N_pad, L_pad), jnp.float32),
                   jax.ShapeDtypeStruct((N_pad, L_pad), jnp.float32),
                   jax.ShapeDtypeStruct((N_pad, 2, L_pad), jnp.float32),
                   jax.ShapeDtypeStruct((N_pad, L_pad), jnp.float32),
                   jax.ShapeDtypeStruct((N_pad, 2, L_pad), jnp.float32)),
        grid=grid,
        in_specs=[pl.BlockSpec((tile_n, D), lambda i: (i, 0)),
                  inv((D, H)), inv((1, H)), inv((H, W2)), inv((1, W2)),
                  inv((D, H)), inv((1, H)), inv((H, W2)), inv((1, W2))],
        out_specs=(row(L_pad), row(L_pad), row3, row(L_pad), row3),
        compiler_params=pltpu.CompilerParams(
            dimension_semantics=("parallel",),
            vmem_limit_bytes=64 * 1024 * 1024),
    )(Xp, *enc_ops, *ctx_ops)

    if N_pad != N:
        mu, sigma, ce, cmu, cc = mu[:N], sigma[:N], ce[:N], cmu[:N], cc[:N]
    if L_pad != L:
        mu, sigma, cmu = mu[:, :L], sigma[:, :L], cmu[:, :L]
    return mu, sigma, ce, cmu, cc


def _kl_gather_kernel(ui_ref, uj_ref, enc_ref, ctx_ref, kl_ref,
                      gi_ref, gj_ref, *, M, L_pad, chunk):
    base = pl.program_id(0) * M

    def recip(x):
        r = pl.reciprocal(x, approx=True)
        return r * (2.0 - x * r)

    def gather(c):
        lo = c * chunk
        for b in range(chunk // 8):
            lob = lo + b * 8
            vi = [enc_ref[ui_ref[base + lob + k]] for k in range(8)]
            vj = [ctx_ref[uj_ref[base + lob + k]] for k in range(8)]
            for k in range(8):
                gi_ref[lob + k] = vi[k]
                gj_ref[lob + k] = vj[k]

    def compute(c):
        lo = c * chunk
        mu_i = gi_ref[lo:lo + chunk, 0, :]
        sig_i = gi_ref[lo:lo + chunk, 1, :]
        mu_j = gj_ref[lo:lo + chunk, 0, :]
        sig_j = gj_ref[lo:lo + chunk, 1, :]
        d2 = (mu_i - mu_j) ** 2
        t = (sig_j + d2) * recip(sig_i) + (sig_i + d2) * recip(sig_j)
        kl_ref[lo:lo + chunk, :] = (0.25 * jnp.sum(t, axis=1, keepdims=True)
                                    - 0.5 * float(L_pad))

    n_chunks = M // chunk
    gather(0)
    for c in range(1, n_chunks):
        gather(c)
        compute(c - 1)
    compute(n_chunks - 1)


def _kl_distance(enc_comb, ctx_comb, u_i, u_j, *, tile_b=1024, chunk=128):
    N, _, L_pad = enc_comb.shape
    B = u_i.shape[0]
    B_pad = _round_up(B, tile_b)
    if B_pad != B:
        u_i = jnp.pad(u_i, (0, B_pad - B))
        u_j = jnp.pad(u_j, (0, B_pad - B))

    enc3 = enc_comb
    ctx3 = ctx_comb
    grid = (B_pad // tile_b,)
    inv3 = pl.BlockSpec((N, 2, L_pad), lambda i, *_: (0, 0, 0))
    out = pl.pallas_call(
        functools.partial(_kl_gather_kernel, M=tile_b, L_pad=L_pad,
                          chunk=chunk),
        out_shape=jax.ShapeDtypeStruct((B_pad, 1), jnp.float32),
        grid_spec=pltpu.PrefetchScalarGridSpec(
            num_scalar_prefetch=2,
            grid=grid,
            in_specs=[inv3, inv3],
            out_specs=pl.BlockSpec((tile_b, 1), lambda i, *_: (i, 0)),
            scratch_shapes=[
                pltpu.VMEM((tile_b, 2, L_pad), jnp.float32),
                pltpu.VMEM((tile_b, 2, L_pad), jnp.float32),
            ],
        ),
        compiler_params=pltpu.CompilerParams(
            dimension_semantics=("parallel",),
            vmem_limit_bytes=100 * 1024 * 1024),
    )(u_i, u_j, enc3, ctx3)
    return out[:B, 0]


@jax.jit
def _glace(X_dense, enc_W1, enc_b1, enc_Wmu, enc_bmu, enc_Wsg, enc_bsg,
           ctx_W1, ctx_b1, ctx_Wmu, ctx_bmu, ctx_Wsg, ctx_bsg, u_i, u_j):
    mu, sigma, enc_comb, ctx_mu, ctx_comb = _encode_both(
        X_dense,
        (enc_W1, enc_b1, enc_Wmu, enc_bmu, enc_Wsg, enc_bsg),
        (ctx_W1, ctx_b1, ctx_Wmu, ctx_bmu, ctx_Wsg, ctx_bsg))
    kl = _kl_distance(enc_comb, ctx_comb, u_i, u_j)
    return kl, mu, sigma, ctx_mu


def kernel(X_dense, enc_W1, enc_b1, enc_Wmu, enc_bmu, enc_Wsg, enc_bsg,
           ctx_W1, ctx_b1, ctx_Wmu, ctx_bmu, ctx_Wsg, ctx_bsg, u_i, u_j):
    return _glace(X_dense, enc_W1, enc_b1, enc_Wmu, enc_bmu, enc_Wsg,
                  enc_bsg, ctx_W1, ctx_b1, ctx_Wmu, ctx_bmu, ctx_Wsg,
                  ctx_bsg, u_i, u_j)


# speedup vs baseline: 1.0150x; 1.0150x over previous
import functools

import jax
import jax.numpy as jnp
from jax.experimental import pallas as pl
from jax.experimental.pallas import tpu as pltpu


def _round_up(x, m):
    return ((x + m - 1) // m) * m


def _encode2_kernel(x_ref, w1e_ref, b1e_ref, whe_ref, bhe_ref,
                    w1c_ref, b1c_ref, whc_ref, bhc_ref,
                    mu_ref, sg_ref, ce_ref, cmu_ref, cc_ref, *, L_pad):
    x = x_ref[...].astype(jnp.bfloat16)

    def heads(w1_ref, b1_ref, wh_ref, bh_ref):
        h = jnp.dot(x, w1_ref[...], preferred_element_type=jnp.float32)
        h = jnp.maximum(h + b1_ref[...], 0.0).astype(jnp.bfloat16)
        head = jnp.dot(h, wh_ref[...], preferred_element_type=jnp.float32)
        head = head + bh_ref[...]
        mu = head[:, :L_pad]
        sg = head[:, L_pad:]
        elu = jnp.where(sg > 0, sg, jnp.exp(jnp.minimum(sg, 0.0)) - 1.0)
        return mu, elu + 1.0 + 1e-14

    mu_e, sig_e = heads(w1e_ref, b1e_ref, whe_ref, bhe_ref)
    mu_ref[...] = mu_e
    sg_ref[...] = sig_e
    ce_ref[:, 0, :] = mu_e
    ce_ref[:, 1, :] = sig_e

    mu_c, sig_c = heads(w1c_ref, b1c_ref, whc_ref, bhc_ref)
    cmu_ref[...] = mu_c
    cc_ref[:, 0, :] = mu_c
    cc_ref[:, 1, :] = sig_c


def _encode_both(X, enc_w, ctx_w, *, tile_n=1024):
    N, D = X.shape
    H = enc_w[0].shape[1]
    L = enc_w[2].shape[1]
    L_pad = _round_up(L, 128)
    W2 = 2 * L_pad

    def head_operands(W1, b1, Wmu, bmu, Wsg, bsg):
        Whead = jnp.zeros((H, W2), jnp.float32)
        Whead = Whead.at[:, :L].set(Wmu)
        Whead = Whead.at[:, L_pad:L_pad + L].set(Wsg)
        bhead = jnp.zeros((1, W2), jnp.float32)
        bhead = bhead.at[0, :L].set(bmu)
        bhead = bhead.at[0, L_pad:L_pad + L].set(bsg)
        return (W1.astype(jnp.bfloat16), b1.reshape(1, H),
                Whead.astype(jnp.bfloat16), bhead)

    enc_ops = head_operands(*enc_w)
    ctx_ops = head_operands(*ctx_w)

    tile_n = min(tile_n, _round_up(N, 8))
    N_pad = _round_up(N, tile_n)
    Xp = X if N_pad == N else jnp.pad(X, ((0, N_pad - N), (0, 0)))

    grid = (N_pad // tile_n,)
    inv = lambda shape: pl.BlockSpec(shape, lambda i: (0, 0))
    row = lambda w: pl.BlockSpec((tile_n, w), lambda i: (i, 0))

    row3 = pl.BlockSpec((tile_n, 2, L_pad), lambda i: (i, 0, 0))
    mu, sigma, ce, cmu, cc = pl.pallas_call(
        functools.partial(_encode2_kernel, L_pad=L_pad),
        out_shape=(jax.ShapeDtypeStruct((N_pad, L_pad), jnp.float32),
                   jax.ShapeDtypeStruct((N_pad, L_pad), jnp.float32),
                   jax.ShapeDtypeStruct((N_pad, 2, L_pad), jnp.float32),
                   jax.ShapeDtypeStruct((N_pad, L_pad), jnp.float32),
                   jax.ShapeDtypeStruct((N_pad, 2, L_pad), jnp.float32)),
        grid=grid,
        in_specs=[pl.BlockSpec((tile_n, D), lambda i: (i, 0)),
                  inv((D, H)), inv((1, H)), inv((H, W2)), inv((1, W2)),
                  inv((D, H)), inv((1, H)), inv((H, W2)), inv((1, W2))],
        out_specs=(row(L_pad), row(L_pad), row3, row(L_pad), row3),
        compiler_params=pltpu.CompilerParams(
            dimension_semantics=("parallel",),
            vmem_limit_bytes=64 * 1024 * 1024),
    )(Xp, *enc_ops, *ctx_ops)

    if N_pad != N:
        mu, sigma, ce, cmu, cc = mu[:N], sigma[:N], ce[:N], cmu[:N], cc[:N]
    if L_pad != L:
        mu, sigma, cmu = mu[:, :L], sigma[:, :L], cmu[:, :L]
    return mu, sigma, ce, cmu, cc


def _kl_gather_kernel(ui_ref, uj_ref, enc_ref, ctx_ref, kl_ref,
                      gi_ref, gj_ref, *, M, L_pad, chunk):
    base = pl.program_id(0) * M

    def recip(x):
        r = pl.reciprocal(x, approx=True)
        return r * (2.0 - x * r)

    def gather(c):
        lo = c * chunk
        for b in range(chunk // 8):
            lob = lo + b * 8
            vi = [enc_ref[ui_ref[base + lob + k]] for k in range(8)]
            vj = [ctx_ref[uj_ref[base + lob + k]] for k in range(8)]
            for k in range(8):
                gi_ref[lob + k] = vi[k]
                gj_ref[lob + k] = vj[k]

    def compute(c):
        lo = c * chunk
        mu_i = gi_ref[lo:lo + chunk, 0, :]
        sig_i = gi_ref[lo:lo + chunk, 1, :]
        mu_j = gj_ref[lo:lo + chunk, 0, :]
        sig_j = gj_ref[lo:lo + chunk, 1, :]
        d2 = (mu_i - mu_j) ** 2
        t = (sig_j + d2) * recip(sig_i) + (sig_i + d2) * recip(sig_j)
        kl_ref[lo:lo + chunk, :] = (0.25 * jnp.sum(t, axis=1, keepdims=True)
                                    - 0.5 * float(L_pad))

    n_chunks = M // chunk
    gather(0)
    for c in range(1, n_chunks):
        gather(c)
        compute(c - 1)
    compute(n_chunks - 1)


def _kl_distance(enc_comb, ctx_comb, u_i, u_j, *, tile_b=1024, chunk=128):
    N, _, L_pad = enc_comb.shape
    B = u_i.shape[0]
    B_pad = _round_up(B, tile_b)
    if B_pad != B:
        u_i = jnp.pad(u_i, (0, B_pad - B))
        u_j = jnp.pad(u_j, (0, B_pad - B))

    enc3 = enc_comb
    ctx3 = ctx_comb
    grid = (B_pad // tile_b,)
    inv3 = pl.BlockSpec((N, 2, L_pad), lambda i, *_: (0, 0, 0))
    out = pl.pallas_call(
        functools.partial(_kl_gather_kernel, M=tile_b, L_pad=L_pad,
                          chunk=chunk),
        out_shape=jax.ShapeDtypeStruct((B_pad, 1), jnp.float32),
        grid_spec=pltpu.PrefetchScalarGridSpec(
            num_scalar_prefetch=2,
            grid=grid,
            in_specs=[inv3, inv3],
            out_specs=pl.BlockSpec((tile_b, 1), lambda i, *_: (i, 0)),
            scratch_shapes=[
                pltpu.VMEM((tile_b, 2, L_pad), jnp.float32),
                pltpu.VMEM((tile_b, 2, L_pad), jnp.float32),
            ],
        ),
        compiler_params=pltpu.CompilerParams(
            dimension_semantics=("parallel",),
            vmem_limit_bytes=100 * 1024 * 1024),
    )(u_i, u_j, enc3, ctx3)
    return out[:B, 0]


@jax.jit
def _glace(X_dense, enc_W1, enc_b1, enc_Wmu, enc_bmu, enc_Wsg, enc_bsg,
           ctx_W1, ctx_b1, ctx_Wmu, ctx_bmu, ctx_Wsg, ctx_bsg, u_i, u_j):
    mu, sigma, enc_comb, ctx_mu, ctx_comb = _encode_both(
        X_dense,
        (enc_W1, enc_b1, enc_Wmu, enc_bmu, enc_Wsg, enc_bsg),
        (ctx_W1, ctx_b1, ctx_Wmu, ctx_bmu, ctx_Wsg, ctx_bsg))
    kl = _kl_distance(enc_comb, ctx_comb, u_i, u_j)
    return kl, mu, sigma, ctx_mu


def kernel(X_dense, enc_W1, enc_b1, enc_Wmu, enc_bmu, enc_Wsg, enc_bsg,
           ctx_W1, ctx_b1, ctx_Wmu, ctx_bmu, ctx_Wsg, ctx_bsg, u_i, u_j):
    return _glace(X_dense, enc_W1, enc_b1, enc_Wmu, enc_bmu, enc_Wsg,
                  enc_bsg, ctx_W1, ctx_b1, ctx_Wmu, ctx_bmu, ctx_Wsg,
                  ctx_bsg, u_i, u_j)


# speedup vs baseline: 1.0249x; 1.0098x over previous
import functools

import jax
import jax.numpy as jnp
from jax.experimental import pallas as pl
from jax.experimental.pallas import tpu as pltpu


def _round_up(x, m):
    return ((x + m - 1) // m) * m


def _encode2_kernel(x_ref, w1e_ref, b1e_ref, whe_ref, bhe_ref,
                    w1c_ref, b1c_ref, whc_ref, bhc_ref,
                    mu_ref, sg_ref, ce_ref, cmu_ref, cc_ref, *, L_pad):
    x = x_ref[...].astype(jnp.bfloat16)

    def heads(w1_ref, b1_ref, wh_ref, bh_ref):
        h = jnp.dot(x, w1_ref[...], preferred_element_type=jnp.float32)
        h = jnp.maximum(h + b1_ref[...], 0.0).astype(jnp.bfloat16)
        head = jnp.dot(h, wh_ref[...], preferred_element_type=jnp.float32)
        head = head + bh_ref[...]
        mu = head[:, :L_pad]
        sg = head[:, L_pad:]
        elu = jnp.where(sg > 0, sg, jnp.exp(jnp.minimum(sg, 0.0)) - 1.0)
        return mu, elu + 1.0 + 1e-14

    mu_e, sig_e = heads(w1e_ref, b1e_ref, whe_ref, bhe_ref)
    mu_ref[...] = mu_e
    sg_ref[...] = sig_e
    ce_ref[:, 0, :] = mu_e
    ce_ref[:, 1, :] = sig_e

    mu_c, sig_c = heads(w1c_ref, b1c_ref, whc_ref, bhc_ref)
    cmu_ref[...] = mu_c
    cc_ref[:, 0, :] = mu_c
    cc_ref[:, 1, :] = sig_c


def _encode_both(X, enc_w, ctx_w, *, tile_n=1024):
    N, D = X.shape
    H = enc_w[0].shape[1]
    L = enc_w[2].shape[1]
    L_pad = _round_up(L, 128)
    W2 = 2 * L_pad

    def head_operands(W1, b1, Wmu, bmu, Wsg, bsg):
        Whead = jnp.zeros((H, W2), jnp.float32)
        Whead = Whead.at[:, :L].set(Wmu)
        Whead = Whead.at[:, L_pad:L_pad + L].set(Wsg)
        bhead = jnp.zeros((1, W2), jnp.float32)
        bhead = bhead.at[0, :L].set(bmu)
        bhead = bhead.at[0, L_pad:L_pad + L].set(bsg)
        return (W1.astype(jnp.bfloat16), b1.reshape(1, H),
                Whead.astype(jnp.bfloat16), bhead)

    enc_ops = head_operands(*enc_w)
    ctx_ops = head_operands(*ctx_w)

    tile_n = min(tile_n, _round_up(N, 8))
    N_pad = _round_up(N, tile_n)
    Xp = X if N_pad == N else jnp.pad(X, ((0, N_pad - N), (0, 0)))

    grid = (N_pad // tile_n,)
    inv = lambda shape: pl.BlockSpec(shape, lambda i: (0, 0))
    row = lambda w: pl.BlockSpec((tile_n, w), lambda i: (i, 0))

    row3 = pl.BlockSpec((tile_n, 2, L_pad), lambda i: (i, 0, 0))
    mu, sigma, ce, cmu, cc = pl.pallas_call(
        functools.partial(_encode2_kernel, L_pad=L_pad),
        out_shape=(jax.ShapeDtypeStruct((N_pad, L_pad), jnp.float32),
                   jax.ShapeDtypeStruct((N_pad, L_pad), jnp.float32),
                   jax.ShapeDtypeStruct((N_pad, 2, L_pad), jnp.float32),
                   jax.ShapeDtypeStruct((N_pad, L_pad), jnp.float32),
                   jax.ShapeDtypeStruct((N_pad, 2, L_pad), jnp.float32)),
        grid=grid,
        in_specs=[pl.BlockSpec((tile_n, D), lambda i: (i, 0)),
                  inv((D, H)), inv((1, H)), inv((H, W2)), inv((1, W2)),
                  inv((D, H)), inv((1, H)), inv((H, W2)), inv((1, W2))],
        out_specs=(row(L_pad), row(L_pad), row3, row(L_pad), row3),
        compiler_params=pltpu.CompilerParams(
            dimension_semantics=("parallel",),
            vmem_limit_bytes=64 * 1024 * 1024),
    )(Xp, *enc_ops, *ctx_ops)

    if N_pad != N:
        mu, sigma, ce, cmu, cc = mu[:N], sigma[:N], ce[:N], cmu[:N], cc[:N]
    if L_pad != L:
        mu, sigma, cmu = mu[:, :L], sigma[:, :L], cmu[:, :L]
    return mu, sigma, ce, cmu, cc


def _kl_gather_kernel(ui_ref, uj_ref, enc_ref, ctx_ref, kl_ref,
                      gi_ref, gj_ref, *, M, L_pad, chunk):
    base = pl.program_id(0) * M

    def recip(x):
        r = pl.reciprocal(x, approx=True)
        return r * (2.0 - x * r)

    def gather(c):
        lo = c * chunk
        for b in range(chunk // 8):
            lob = lo + b * 8
            vi = [enc_ref[ui_ref[base + lob + k]] for k in range(8)]
            vj = [ctx_ref[uj_ref[base + lob + k]] for k in range(8)]
            for k in range(8):
                gi_ref[lob + k] = vi[k]
                gj_ref[lob + k] = vj[k]

    def compute(c):
        lo = c * chunk
        mu_i = gi_ref[lo:lo + chunk, 0, :]
        sig_i = gi_ref[lo:lo + chunk, 1, :]
        mu_j = gj_ref[lo:lo + chunk, 0, :]
        sig_j = gj_ref[lo:lo + chunk, 1, :]
        d2 = (mu_i - mu_j) ** 2
        t = (sig_j + d2) * recip(sig_i) + (sig_i + d2) * recip(sig_j)
        kl_ref[lo:lo + chunk, :] = (0.25 * jnp.sum(t, axis=1, keepdims=True)
                                    - 0.5 * float(L_pad))

    n_chunks = M // chunk
    gather(0)
    for c in range(1, n_chunks):
        gather(c)
        compute(c - 1)
    compute(n_chunks - 1)


def _kl_distance(enc_comb, ctx_comb, u_i, u_j, *, tile_b=2048, chunk=128):
    N, _, L_pad = enc_comb.shape
    B = u_i.shape[0]
    B_pad = _round_up(B, tile_b)
    if B_pad != B:
        u_i = jnp.pad(u_i, (0, B_pad - B))
        u_j = jnp.pad(u_j, (0, B_pad - B))

    enc3 = enc_comb
    ctx3 = ctx_comb
    grid = (B_pad // tile_b,)
    inv3 = pl.BlockSpec((N, 2, L_pad), lambda i, *_: (0, 0, 0))
    out = pl.pallas_call(
        functools.partial(_kl_gather_kernel, M=tile_b, L_pad=L_pad,
                          chunk=chunk),
        out_shape=jax.ShapeDtypeStruct((B_pad, 1), jnp.float32),
        grid_spec=pltpu.PrefetchScalarGridSpec(
            num_scalar_prefetch=2,
            grid=grid,
            in_specs=[inv3, inv3],
            out_specs=pl.BlockSpec((tile_b, 1), lambda i, *_: (i, 0)),
            scratch_shapes=[
                pltpu.VMEM((tile_b, 2, L_pad), jnp.float32),
                pltpu.VMEM((tile_b, 2, L_pad), jnp.float32),
            ],
        ),
        compiler_params=pltpu.CompilerParams(
            dimension_semantics=("parallel",),
            vmem_limit_bytes=100 * 1024 * 1024),
    )(u_i, u_j, enc3, ctx3)
    return out[:B, 0]


@jax.jit
def _glace(X_dense, enc_W1, enc_b1, enc_Wmu, enc_bmu, enc_Wsg, enc_bsg,
           ctx_W1, ctx_b1, ctx_Wmu, ctx_bmu, ctx_Wsg, ctx_bsg, u_i, u_j):
    mu, sigma, enc_comb, ctx_mu, ctx_comb = _encode_both(
        X_dense,
        (enc_W1, enc_b1, enc_Wmu, enc_bmu, enc_Wsg, enc_bsg),
        (ctx_W1, ctx_b1, ctx_Wmu, ctx_bmu, ctx_Wsg, ctx_bsg))
    kl = _kl_distance(enc_comb, ctx_comb, u_i, u_j)
    return kl, mu, sigma, ctx_mu


def kernel(X_dense, enc_W1, enc_b1, enc_Wmu, enc_bmu, enc_Wsg, enc_bsg,
           ctx_W1, ctx_b1, ctx_Wmu, ctx_bmu, ctx_Wsg, ctx_bsg, u_i, u_j):
    return _glace(X_dense, enc_W1, enc_b1, enc_Wmu, enc_bmu, enc_Wsg,
                  enc_bsg, ctx_W1, ctx_b1, ctx_Wmu, ctx_bmu, ctx_Wsg,
                  ctx_bsg, u_i, u_j)


# speedup vs baseline: 1.0284x; 1.0034x over previous
import functools

import jax
import jax.numpy as jnp
from jax.experimental import pallas as pl
from jax.experimental.pallas import tpu as pltpu


def _round_up(x, m):
    return ((x + m - 1) // m) * m


def _encode2_kernel(x_ref, w1e_ref, b1e_ref, whe_ref, bhe_ref,
                    w1c_ref, b1c_ref, whc_ref, bhc_ref,
                    mu_ref, sg_ref, ce_ref, cmu_ref, cc_ref, *, L_pad):
    x = x_ref[...].astype(jnp.bfloat16)

    def heads(w1_ref, b1_ref, wh_ref, bh_ref):
        h = jnp.dot(x, w1_ref[...], preferred_element_type=jnp.float32)
        h = jnp.maximum(h + b1_ref[...], 0.0).astype(jnp.bfloat16)
        head = jnp.dot(h, wh_ref[...], preferred_element_type=jnp.float32)
        head = head + bh_ref[...]
        mu = head[:, :L_pad]
        sg = head[:, L_pad:]
        elu = jnp.where(sg > 0, sg, jnp.exp(jnp.minimum(sg, 0.0)) - 1.0)
        return mu, elu + 1.0 + 1e-14

    mu_e, sig_e = heads(w1e_ref, b1e_ref, whe_ref, bhe_ref)
    mu_ref[...] = mu_e
    sg_ref[...] = sig_e
    ce_ref[:, 0, :] = mu_e
    ce_ref[:, 1, :] = sig_e

    mu_c, sig_c = heads(w1c_ref, b1c_ref, whc_ref, bhc_ref)
    cmu_ref[...] = mu_c
    cc_ref[:, 0, :] = mu_c
    cc_ref[:, 1, :] = sig_c


def _encode_both(X, enc_w, ctx_w, *, tile_n=1024):
    N, D = X.shape
    H = enc_w[0].shape[1]
    L = enc_w[2].shape[1]
    L_pad = _round_up(L, 128)
    W2 = 2 * L_pad

    def head_operands(W1, b1, Wmu, bmu, Wsg, bsg):
        Whead = jnp.zeros((H, W2), jnp.float32)
        Whead = Whead.at[:, :L].set(Wmu)
        Whead = Whead.at[:, L_pad:L_pad + L].set(Wsg)
        bhead = jnp.zeros((1, W2), jnp.float32)
        bhead = bhead.at[0, :L].set(bmu)
        bhead = bhead.at[0, L_pad:L_pad + L].set(bsg)
        return (W1.astype(jnp.bfloat16), b1.reshape(1, H),
                Whead.astype(jnp.bfloat16), bhead)

    enc_ops = head_operands(*enc_w)
    ctx_ops = head_operands(*ctx_w)

    tile_n = min(tile_n, _round_up(N, 8))
    N_pad = _round_up(N, tile_n)
    Xp = X if N_pad == N else jnp.pad(X, ((0, N_pad - N), (0, 0)))

    grid = (N_pad // tile_n,)
    inv = lambda shape: pl.BlockSpec(shape, lambda i: (0, 0))
    row = lambda w: pl.BlockSpec((tile_n, w), lambda i: (i, 0))

    row3 = pl.BlockSpec((tile_n, 2, L_pad), lambda i: (i, 0, 0))
    mu, sigma, ce, cmu, cc = pl.pallas_call(
        functools.partial(_encode2_kernel, L_pad=L_pad),
        out_shape=(jax.ShapeDtypeStruct((N_pad, L_pad), jnp.float32),
                   jax.ShapeDtypeStruct((N_pad, L_pad), jnp.float32),
                   jax.ShapeDtypeStruct((N_pad, 2, L_pad), jnp.float32),
                   jax.ShapeDtypeStruct((N_pad, L_pad), jnp.float32),
                   jax.ShapeDtypeStruct((N_pad, 2, L_pad), jnp.float32)),
        grid=grid,
        in_specs=[pl.BlockSpec((tile_n, D), lambda i: (i, 0)),
                  inv((D, H)), inv((1, H)), inv((H, W2)), inv((1, W2)),
                  inv((D, H)), inv((1, H)), inv((H, W2)), inv((1, W2))],
        out_specs=(row(L_pad), row(L_pad), row3, row(L_pad), row3),
        compiler_params=pltpu.CompilerParams(
            dimension_semantics=("parallel",),
            vmem_limit_bytes=64 * 1024 * 1024),
    )(Xp, *enc_ops, *ctx_ops)

    if N_pad != N:
        mu, sigma, ce, cmu, cc = mu[:N], sigma[:N], ce[:N], cmu[:N], cc[:N]
    if L_pad != L:
        mu, sigma, cmu = mu[:, :L], sigma[:, :L], cmu[:, :L]
    return mu, sigma, ce, cmu, cc


def _kl_gather_kernel(ui_ref, uj_ref, enc_ref, ctx_ref, kl_ref,
                      gi_ref, gj_ref, *, M, L_pad, chunk):
    base = pl.program_id(0) * M

    def recip(x):
        r = pl.reciprocal(x, approx=True)
        return r * (2.0 - x * r)

    def gather(c):
        lo = c * chunk
        for b in range(chunk // 8):
            lob = lo + b * 8
            vi = [enc_ref[ui_ref[base + lob + k]] for k in range(8)]
            vj = [ctx_ref[uj_ref[base + lob + k]] for k in range(8)]
            for k in range(8):
                gi_ref[lob + k] = vi[k]
                gj_ref[lob + k] = vj[k]

    def compute(c):
        lo = c * chunk
        mu_i = gi_ref[lo:lo + chunk, 0, :]
        sig_i = gi_ref[lo:lo + chunk, 1, :]
        mu_j = gj_ref[lo:lo + chunk, 0, :]
        sig_j = gj_ref[lo:lo + chunk, 1, :]
        d2 = (mu_i - mu_j) ** 2
        t = (sig_j + d2) * recip(sig_i) + (sig_i + d2) * recip(sig_j)
        kl_ref[lo:lo + chunk, :] = (0.25 * jnp.sum(t, axis=1, keepdims=True)
                                    - 0.5 * float(L_pad))

    n_chunks = M // chunk
    gather(0)
    for c in range(1, n_chunks):
        gather(c)
        compute(c - 1)
    compute(n_chunks - 1)


def _kl_distance(enc_comb, ctx_comb, u_i, u_j, *, tile_b=4096, chunk=128):
    N, _, L_pad = enc_comb.shape
    B = u_i.shape[0]
    B_pad = _round_up(B, tile_b)
    if B_pad != B:
        u_i = jnp.pad(u_i, (0, B_pad - B))
        u_j = jnp.pad(u_j, (0, B_pad - B))

    enc3 = enc_comb
    ctx3 = ctx_comb
    grid = (B_pad // tile_b,)
    inv3 = pl.BlockSpec((N, 2, L_pad), lambda i, *_: (0, 0, 0))
    out = pl.pallas_call(
        functools.partial(_kl_gather_kernel, M=tile_b, L_pad=L_pad,
                          chunk=chunk),
        out_shape=jax.ShapeDtypeStruct((B_pad, 1), jnp.float32),
        grid_spec=pltpu.PrefetchScalarGridSpec(
            num_scalar_prefetch=2,
            grid=grid,
            in_specs=[inv3, inv3],
            out_specs=pl.BlockSpec((tile_b, 1), lambda i, *_: (i, 0)),
            scratch_shapes=[
                pltpu.VMEM((tile_b, 2, L_pad), jnp.float32),
                pltpu.VMEM((tile_b, 2, L_pad), jnp.float32),
            ],
        ),
        compiler_params=pltpu.CompilerParams(
            dimension_semantics=("parallel",),
            vmem_limit_bytes=100 * 1024 * 1024),
    )(u_i, u_j, enc3, ctx3)
    return out[:B, 0]


@jax.jit
def _glace(X_dense, enc_W1, enc_b1, enc_Wmu, enc_bmu, enc_Wsg, enc_bsg,
           ctx_W1, ctx_b1, ctx_Wmu, ctx_bmu, ctx_Wsg, ctx_bsg, u_i, u_j):
    mu, sigma, enc_comb, ctx_mu, ctx_comb = _encode_both(
        X_dense,
        (enc_W1, enc_b1, enc_Wmu, enc_bmu, enc_Wsg, enc_bsg),
        (ctx_W1, ctx_b1, ctx_Wmu, ctx_bmu, ctx_Wsg, ctx_bsg))
    kl = _kl_distance(enc_comb, ctx_comb, u_i, u_j)
    return kl, mu, sigma, ctx_mu


def kernel(X_dense, enc_W1, enc_b1, enc_Wmu, enc_bmu, enc_Wsg, enc_bsg,
           ctx_W1, ctx_b1, ctx_Wmu, ctx_bmu, ctx_Wsg, ctx_bsg, u_i, u_j):
    return _glace(X_dense, enc_W1, enc_b1, enc_Wmu, enc_bmu, enc_Wsg,
                  enc_bsg, ctx_W1, ctx_b1, ctx_Wmu, ctx_bmu, ctx_Wsg,
                  ctx_bsg, u_i, u_j)
